# Initial kernel scaffold; baseline (speedup 1.0000x reference)
import jax
import jax.numpy as jnp
from jax.experimental import pallas as pl
from jax.experimental.pallas import tpu as pltpu

_D = 4


def _make_kernel(B):
    R1 = B * 28
    R2 = B * 14

    def body(x_ref, a1_ref, b1t_ref, a2_ref, b2t_ref, wlp_ref, blt_ref,
             feat_ref, logit_ref, xp1, xc1, xp2, xc2):
        f32 = jnp.float32
        bf16 = jnp.bfloat16

        xp1[...] = jnp.zeros_like(xp1)
        xp1[_D:_D + R1, 2:30] = x_ref[...].reshape(R1, 28)

        h1 = jax.lax.broadcasted_iota(jnp.int32, (R1, 1), 0) % 28
        for i in range(5):
            d = i - 2
            s = xp1[_D + d:_D + d + R1, :]
            if d < 0:
                s = jnp.where(h1 >= -d, s, jnp.bfloat16(0))
            elif d > 0:
                s = jnp.where(h1 <= 27 - d, s, jnp.bfloat16(0))
            xc1[:, 32 * i:32 * i + 32] = s

        acc1 = jnp.dot(xc1[...], a1_ref[...], preferred_element_type=f32)
        y1 = jnp.maximum(acc1 + b1t_ref[...], 0.0)

        hp = jnp.maximum(y1[0::2], y1[1::2])
        xp2[...] = jnp.zeros_like(xp2)
        for q in range(14):
            xp2[_D:_D + R2, 32 + 16 * q:48 + 16 * q] = jnp.maximum(
                hp[:, 32 * q:32 * q + 16],
                hp[:, 32 * q + 16:32 * q + 32]).astype(bf16)

        h2 = jax.lax.broadcasted_iota(jnp.int32, (R2, 1), 0) % 14
        for i in range(5):
            d = i - 2
            s = xp2[_D + d:_D + d + R2, :]
            if d < 0:
                s = jnp.where(h2 >= -d, s, jnp.bfloat16(0))
            elif d > 0:
                s = jnp.where(h2 <= 13 - d, s, jnp.bfloat16(0))
            xc2[:, 288 * i:288 * i + 288] = s

        acc2 = jnp.dot(xc2[...], a2_ref[...], preferred_element_type=f32)
        y2 = jnp.maximum(acc2 + b2t_ref[...], 0.0)

        hq = jnp.maximum(y2[0::2], y2[1::2])
        parts = [jnp.maximum(hq[:, 64 * q:64 * q + 32],
                             hq[:, 64 * q + 32:64 * q + 64])
                 for q in range(7)]
        fv = jnp.concatenate(parts, axis=1)
        feat_ref[...] = fv.reshape(B, 7, 224)

        fb = fv.astype(bf16)
        acc = jnp.dot(fb[0::7], wlp_ref[0], preferred_element_type=f32)
        for h in range(1, 7):
            acc = acc + jnp.dot(fb[h::7], wlp_ref[h],
                                preferred_element_type=f32)
        logit_ref[...] = acc + blt_ref[...]

    return body, R1, R2


def _forward(x2d, a1c, b1t, a2c, b2t, wlp, blt):
    n = x2d.shape[0]
    B = 128 if n % 128 == 0 else (64 if n % 64 == 0 else n)
    body, R1, R2 = _make_kernel(B)
    bf16 = jnp.bfloat16

    feat_k, logit_k = pl.pallas_call(
        body,
        out_shape=(jax.ShapeDtypeStruct((n, 7, 224), jnp.float32),
                   jax.ShapeDtypeStruct((n, 128), jnp.float32)),
        grid=(n // B,),
        in_specs=[
            pl.BlockSpec((B, 28, 28), lambda i: (i, 0, 0)),
            pl.BlockSpec((160, 448), lambda i: (0, 0)),
            pl.BlockSpec((1, 448), lambda i: (0, 0)),
            pl.BlockSpec((1440, 448), lambda i: (0, 0)),
            pl.BlockSpec((1, 448), lambda i: (0, 0)),
            pl.BlockSpec((7, 224, 128), lambda i: (0, 0, 0)),
            pl.BlockSpec((1, 128), lambda i: (0, 0)),
        ],
        out_specs=(
            pl.BlockSpec((B, 7, 224), lambda i: (i, 0, 0)),
            pl.BlockSpec((B, 128), lambda i: (i, 0)),
        ),
        scratch_shapes=[
            pltpu.VMEM((R1 + 8, 32), bf16),
            pltpu.VMEM((R1, 160), bf16),
            pltpu.VMEM((R2 + 8, 288), bf16),
            pltpu.VMEM((R2, 1440), bf16),
        ],
        compiler_params=pltpu.CompilerParams(
            dimension_semantics=("parallel",),
            vmem_limit_bytes=100 * 1024 * 1024),
    )(x2d, a1c, b1t, a2c, b2t, wlp, blt)
    return feat_k, logit_k


@jax.jit
def kernel(x, a1, b1t, a2, b2t, wlp, blt):
    n = x.shape[0]
    x2d = x.reshape(n, 28, 28).astype(jnp.bfloat16)
    a1c = a1.reshape(160, 448).astype(jnp.bfloat16)
    a2c = a2.reshape(1440, 448).astype(jnp.bfloat16)
    wlpb = wlp.astype(jnp.bfloat16)
    feat_k, logit_k = _forward(x2d, a1c, b1t, a2c, b2t, wlpb, blt)
    feat = feat_k.reshape(n, 7, 7, 32).transpose(0, 3, 1, 2).reshape(n, 1568)
    logits = logit_k[:, :10]
    return logits, feat


# speedup vs baseline: 6.7597x; 6.7597x over previous
import jax
import jax.numpy as jnp
from jax.experimental import pallas as pl
from jax.experimental.pallas import tpu as pltpu

_D = 4


def _make_kernel(B):
    R3 = B * 7

    def body(x0_ref, x1_ref, x2_ref, x3_ref,
             a1_ref, b1t_ref, a2_ref, b2t_ref, wf_ref, blt_ref,
             feat_ref, logit_ref,
             xp0, xp1, xp2, xp3, c10, c11, c12, c13,
             xqe, xqo, c2e, c2o, fsp, xfc):
        f32 = jnp.float32
        bf16 = jnp.bfloat16
        xps = [xp0, xp1, xp2, xp3]
        xrefs = [x0_ref, x1_ref, x2_ref, x3_ref]
        xc1s = [c10, c11, c12, c13]
        h7 = jax.lax.broadcasted_iota(jnp.int32, (R3, 1), 0) % 7

        def shifted(src, e):
            s = src[_D + e:_D + e + R3, :]
            if e < 0:
                s = jnp.where(h7 >= -e, s, jnp.bfloat16(0))
            elif e > 0:
                s = jnp.where(h7 <= 6 - e, s, jnp.bfloat16(0))
            return s

        for m in range(4):
            xps[m][...] = jnp.zeros_like(xps[m])
            xps[m][_D:_D + R3, 2:30] = xrefs[m][...]

        y1 = []
        for m in range(4):
            for i in range(5):
                md = m + i - 2
                xc1s[m][:, 32 * i:32 * i + 32] = shifted(xps[md % 4], md // 4)
            acc = jnp.dot(xc1s[m][...], a1_ref[...], preferred_element_type=f32)
            y1.append(jnp.maximum(acc + b1t_ref[...], 0.0))

        hpe = jnp.maximum(y1[0], y1[1])
        hpo = jnp.maximum(y1[2], y1[3])
        for xq, hp in ((xqe, hpe), (xqo, hpo)):
            xq[...] = jnp.zeros_like(xq)
            for q in range(14):
                xq[_D:_D + R3, 32 + 16 * q:48 + 16 * q] = jnp.maximum(
                    hp[:, 32 * q:32 * q + 16],
                    hp[:, 32 * q + 16:32 * q + 32]).astype(bf16)

        y2 = []
        for v in range(2):
            for i in range(5):
                vd = v + i - 2
                src = xqe if vd % 2 == 0 else xqo
                c2 = c2e if v == 0 else c2o
                c2[:, 288 * i:288 * i + 288] = shifted(src, vd // 2)
            c2 = c2e if v == 0 else c2o
            acc = jnp.dot(c2[...], a2_ref[...], preferred_element_type=f32)
            y2.append(jnp.maximum(acc + b2t_ref[...], 0.0))

        hq = jnp.maximum(y2[0], y2[1])
        fsp[R3:, :] = jnp.zeros_like(fsp[R3:, :])
        for q in range(7):
            part = jnp.maximum(hq[:, 64 * q:64 * q + 32],
                               hq[:, 64 * q + 32:64 * q + 64])
            feat_ref[:, 32 * q:32 * q + 32] = part
            fsp[0:R3, 32 * q:32 * q + 32] = part.astype(bf16)

        for h in range(7):
            xfc[:, 224 * h:224 * h + 224] = fsp[h:h + R3, :]
        acc = jnp.dot(xfc[...], wf_ref[...], preferred_element_type=f32)
        logit_ref[...] = acc + blt_ref[...]

    return body, R3


def _forward(xs, a1c, b1t, a2c, b2t, wf, blt):
    n = xs[0].shape[0] // 7
    B = 128 if n % 128 == 0 else (64 if n % 64 == 0 else n)
    body, R3 = _make_kernel(B)
    bf16 = jnp.bfloat16

    feat_k, logit_k = pl.pallas_call(
        body,
        out_shape=(jax.ShapeDtypeStruct((n * 7, 224), jnp.float32),
                   jax.ShapeDtypeStruct((n * 7, 128), jnp.float32)),
        grid=(n // B,),
        in_specs=[pl.BlockSpec((R3, 28), lambda i: (i, 0))] * 4 + [
            pl.BlockSpec((160, 448), lambda i: (0, 0)),
            pl.BlockSpec((1, 448), lambda i: (0, 0)),
            pl.BlockSpec((1440, 448), lambda i: (0, 0)),
            pl.BlockSpec((1, 448), lambda i: (0, 0)),
            pl.BlockSpec((1568, 128), lambda i: (0, 0)),
            pl.BlockSpec((1, 128), lambda i: (0, 0)),
        ],
        out_specs=(
            pl.BlockSpec((R3, 224), lambda i: (i, 0)),
            pl.BlockSpec((R3, 128), lambda i: (i, 0)),
        ),
        scratch_shapes=(
            [pltpu.VMEM((R3 + 8, 32), bf16)] * 4 +
            [pltpu.VMEM((R3, 160), bf16)] * 4 +
            [pltpu.VMEM((R3 + 8, 288), bf16)] * 2 +
            [pltpu.VMEM((R3, 1440), bf16)] * 2 +
            [pltpu.VMEM((R3 + 8, 224), bf16),
             pltpu.VMEM((R3, 1568), bf16)]
        ),
        compiler_params=pltpu.CompilerParams(
            dimension_semantics=("parallel",),
            vmem_limit_bytes=56 * 1024 * 1024),
    )(*xs, a1c, b1t, a2c, b2t, wf, blt)
    return feat_k, logit_k


@jax.jit
def kernel(x, a1, b1t, a2, b2t, wlp, blt):
    n = x.shape[0]
    x2d = x.reshape(n, 28, 28).astype(jnp.bfloat16)
    xs = [x2d[:, m::4, :].reshape(n * 7, 28) for m in range(4)]
    a1c = a1.reshape(160, 448).astype(jnp.bfloat16)
    a2c = a2.reshape(1440, 448).astype(jnp.bfloat16)
    wf = wlp.reshape(1568, 128).astype(jnp.bfloat16)
    feat_k, logit_k = _forward(xs, a1c, b1t, a2c, b2t, wf, blt)
    feat = feat_k.reshape(n, 7, 7, 32).transpose(0, 3, 1, 2).reshape(n, 1568)
    logits = logit_k[0::7, :10]
    return logits, feat


# speedup vs baseline: 7.7583x; 1.1477x over previous
import jax
import jax.numpy as jnp
from jax.experimental import pallas as pl
from jax.experimental.pallas import tpu as pltpu

_D = 8


def _make_kernel(B):
    R3 = B * 7

    def body(x0_ref, x1_ref, x2_ref, x3_ref,
             a1_ref, b1t_ref, a2_ref, b2t_ref, wf_ref, blt_ref,
             feat_ref, logit_ref,
             xp0, xp1, xp2, xp3, c10, c11, c12, c13,
             xqe, xqo, fsp, xfc):
        f32 = jnp.float32
        bf16 = jnp.bfloat16
        xps = [xp0, xp1, xp2, xp3]
        xrefs = [x0_ref, x1_ref, x2_ref, x3_ref]
        xc1s = [c10, c11, c12, c13]
        h7 = jax.lax.broadcasted_iota(jnp.int32, (R3, 1), 0) % 7
        l448 = jax.lax.broadcasted_iota(jnp.int32, (1, 448), 1)

        def shifted(src, e):
            s = src[_D + e:_D + e + R3, :]
            if e < 0:
                s = jnp.where(h7 >= -e, s, jnp.bfloat16(0))
            elif e > 0:
                s = jnp.where(h7 <= 6 - e, s, jnp.bfloat16(0))
            return s

        for m in range(4):
            xps[m][0:_D, :] = jnp.zeros((_D, 32), bf16)
            xps[m][_D + R3:, :] = jnp.zeros((8, 32), bf16)
            xps[m][_D:_D + R3, 28:32] = jnp.zeros((R3, 4), bf16)
            xps[m][_D:_D + R3, 0:28] = xrefs[m][...]

        y1 = []
        for m in range(4):
            for i in range(5):
                md = m + i - 2
                xc1s[m][:, 32 * i:32 * i + 32] = shifted(xps[md % 4], md // 4)
            acc = jnp.dot(xc1s[m][...], a1_ref[...], preferred_element_type=f32)
            y1.append(jnp.maximum(acc + b1t_ref[...], 0.0))

        hpe = jnp.maximum(y1[0], y1[1])
        hpo = jnp.maximum(y1[2], y1[3])
        for xq, hp in ((xqe, hpe), (xqo, hpo)):
            hps = jnp.concatenate([hp[:, 16:], jnp.zeros((R3, 16), f32)], 1)
            frame = jnp.where(l448 % 32 < 16, jnp.maximum(hp, hps), 0.0)
            xq[0:_D, :] = jnp.zeros((_D, 448), bf16)
            xq[_D + R3:, :] = jnp.zeros((8, 448), bf16)
            xq[_D:_D + R3, :] = frame.astype(bf16)

        y2 = []
        for v in range(2):
            acc = None
            for i in range(5):
                vd = v + i - 2
                s = shifted(xqe if vd % 2 == 0 else xqo, vd // 2)
                p = jnp.dot(s, a2_ref[i], preferred_element_type=f32)
                acc = p if acc is None else acc + p
            y2.append(jnp.maximum(acc + b2t_ref[...], 0.0))

        hq = jnp.maximum(y2[0], y2[1])
        fsp[R3:, :] = jnp.zeros_like(fsp[R3:, :])
        for q in range(7):
            part = jnp.maximum(hq[:, 64 * q:64 * q + 32],
                               hq[:, 64 * q + 32:64 * q + 64])
            feat_ref[:, 32 * q:32 * q + 32] = part
            fsp[0:R3, 32 * q:32 * q + 32] = part.astype(bf16)

        for h in range(7):
            xfc[:, 224 * h:224 * h + 224] = fsp[h:h + R3, :]
        acc = jnp.dot(xfc[...], wf_ref[...], preferred_element_type=f32)
        logit_ref[...] = acc + blt_ref[...]

    return body, R3


def _forward(xs, a1n, b1t, a2f, b2t, wf, blt):
    n = xs[0].shape[0] // 7
    B = 128 if n % 128 == 0 else (64 if n % 64 == 0 else n)
    body, R3 = _make_kernel(B)
    bf16 = jnp.bfloat16

    feat_k, logit_k = pl.pallas_call(
        body,
        out_shape=(jax.ShapeDtypeStruct((n * 7, 224), jnp.float32),
                   jax.ShapeDtypeStruct((n * 7, 128), jnp.float32)),
        grid=(n // B,),
        in_specs=[pl.BlockSpec((R3, 28), lambda i: (i, 0))] * 4 + [
            pl.BlockSpec((160, 448), lambda i: (0, 0)),
            pl.BlockSpec((1, 448), lambda i: (0, 0)),
            pl.BlockSpec((5, 448, 448), lambda i: (0, 0, 0)),
            pl.BlockSpec((1, 448), lambda i: (0, 0)),
            pl.BlockSpec((1568, 128), lambda i: (0, 0)),
            pl.BlockSpec((1, 128), lambda i: (0, 0)),
        ],
        out_specs=(
            pl.BlockSpec((R3, 224), lambda i: (i, 0)),
            pl.BlockSpec((R3, 128), lambda i: (i, 0)),
        ),
        scratch_shapes=(
            [pltpu.VMEM((R3 + 16, 32), bf16)] * 4 +
            [pltpu.VMEM((R3, 160), bf16)] * 4 +
            [pltpu.VMEM((R3 + 16, 448), bf16)] * 2 +
            [pltpu.VMEM((R3 + 8, 224), bf16),
             pltpu.VMEM((R3, 1568), bf16)]
        ),
        compiler_params=pltpu.CompilerParams(
            dimension_semantics=("parallel",),
            vmem_limit_bytes=56 * 1024 * 1024),
    )(*xs, a1n, b1t, a2f, b2t, wf, blt)
    return feat_k, logit_k


@jax.jit
def kernel(x, a1, b1t, a2, b2t, wlp, blt):
    n = x.shape[0]
    x2d = x.reshape(n, 28, 28).astype(jnp.bfloat16)
    xs = [x2d[:, m::4, :].reshape(n * 7, 28) for m in range(4)]
    a1n = jnp.pad(a1[:, 2:30, :], ((0, 0), (0, 4), (0, 0)))
    a1n = a1n.reshape(160, 448).astype(jnp.bfloat16)
    a2f = a2[:, 32:256, :].reshape(5, 14, 16, 448)
    a2f = jnp.pad(a2f, ((0, 0), (0, 0), (0, 16), (0, 0)))
    a2f = a2f.reshape(5, 448, 448).astype(jnp.bfloat16)
    wf = wlp.reshape(1568, 128).astype(jnp.bfloat16)
    feat_k, logit_k = _forward(xs, a1n, b1t, a2f, b2t, wf, blt)
    feat = feat_k.reshape(n, 7, 7, 32).transpose(0, 3, 1, 2).reshape(n, 1568)
    logits = logit_k[0::7, :10]
    return logits, feat


# speedup vs baseline: 9.2743x; 1.1954x over previous
import jax
import jax.numpy as jnp
from jax.experimental import pallas as pl
from jax.experimental.pallas import tpu as pltpu

_D = 8

_SRCS = [(2, -1), (3, -1), (0, 0), (1, 0), (2, 0), (3, 0), (0, 1), (1, 1)]
_MORD = [0, 2, 1, 3]


def _make_kernel(B):
    R3 = B * 7

    def body(x0_ref, x1_ref, x2_ref, x3_ref,
             w1_ref, b1_ref, w2_ref, b2t_ref, wf_ref, blt_ref,
             feat_ref, logit_ref,
             xp0, xp1, xp2, xp3, xcall, xq, fsp):
        f32 = jnp.float32
        bf16 = jnp.bfloat16
        xps = [xp0, xp1, xp2, xp3]
        xrefs = [x0_ref, x1_ref, x2_ref, x3_ref]
        h7 = jax.lax.broadcasted_iota(jnp.int32, (R3, 1), 0) % 7
        l448 = jax.lax.broadcasted_iota(jnp.int32, (1, 448), 1)

        def shifted(src, e):
            s = src[_D + e:_D + e + R3, :]
            if e < 0:
                s = jnp.where(h7 >= -e, s, jnp.bfloat16(0))
            elif e > 0:
                s = jnp.where(h7 <= 6 - e, s, jnp.bfloat16(0))
            return s

        for m in range(4):
            xps[m][0:_D, :] = jnp.zeros((_D, 32), bf16)
            xps[m][_D + R3:, :] = jnp.zeros((8, 32), bf16)
            xps[m][_D:_D + R3, 28:32] = jnp.zeros((R3, 4), bf16)
            xps[m][_D:_D + R3, 0:28] = xrefs[m][...]

        for s, (c, e) in enumerate(_SRCS):
            xcall[:, 32 * s:32 * s + 32] = shifted(xps[c], e)
        acc1 = jnp.dot(xcall[...], w1_ref[...], preferred_element_type=f32)
        ybf = jnp.maximum(acc1 + b1_ref[...], 0.0).astype(bf16)

        hpe = jnp.maximum(ybf[:, 0:448], ybf[:, 896:1344])
        hpo = jnp.maximum(ybf[:, 448:896], ybf[:, 1344:1792])
        z16 = jnp.zeros((R3, 16), bf16)
        me = jnp.maximum(hpe, jnp.concatenate([hpe[:, 16:], z16], 1))
        mo = jnp.maximum(hpo, jnp.concatenate([hpo[:, 16:], z16], 1))
        mo_r = jnp.concatenate([z16, mo[:, :432]], 1)
        xq[0:_D, :] = jnp.zeros((_D, 448), bf16)
        xq[_D + R3:, :] = jnp.zeros((8, 448), bf16)
        xq[_D:_D + R3, :] = jnp.where(l448 % 32 < 16, me, mo_r)

        sl = [shifted(xq, e) for e in (-1, 0, 1)]
        y2 = []
        for v in range(2):
            acc = None
            for ei in range(3):
                p = jnp.dot(sl[ei], w2_ref[3 * v + ei],
                            preferred_element_type=f32)
                acc = p if acc is None else acc + p
            y2.append(jnp.maximum(acc + b2t_ref[...], 0.0))

        hq = jnp.maximum(y2[0], y2[1])
        fsp[R3:, :] = jnp.zeros_like(fsp[R3:, :])
        for q in range(7):
            part = jnp.maximum(hq[:, 64 * q:64 * q + 32],
                               hq[:, 64 * q + 32:64 * q + 64])
            feat_ref[:, 32 * q:32 * q + 32] = part
            fsp[0:R3, 32 * q:32 * q + 32] = part.astype(bf16)

        acc = None
        for h in range(7):
            p = jnp.dot(fsp[h:h + R3, :], wf_ref[h],
                        preferred_element_type=f32)
            acc = p if acc is None else acc + p
        logit_ref[...] = acc + blt_ref[...]

    return body, R3


def _forward(xs, w1, b1, w2, b2t, wf, blt):
    n = xs[0].shape[0] // 7
    B = 128 if n % 128 == 0 else (64 if n % 64 == 0 else n)
    body, R3 = _make_kernel(B)
    bf16 = jnp.bfloat16

    feat_k, logit_k = pl.pallas_call(
        body,
        out_shape=(jax.ShapeDtypeStruct((n * 7, 224), jnp.float32),
                   jax.ShapeDtypeStruct((n * 7, 128), jnp.float32)),
        grid=(n // B,),
        in_specs=[pl.BlockSpec((R3, 28), lambda i: (i, 0))] * 4 + [
            pl.BlockSpec((256, 1792), lambda i: (0, 0)),
            pl.BlockSpec((1, 1792), lambda i: (0, 0)),
            pl.BlockSpec((6, 448, 448), lambda i: (0, 0, 0)),
            pl.BlockSpec((1, 448), lambda i: (0, 0)),
            pl.BlockSpec((7, 224, 128), lambda i: (0, 0, 0)),
            pl.BlockSpec((1, 128), lambda i: (0, 0)),
        ],
        out_specs=(
            pl.BlockSpec((R3, 224), lambda i: (i, 0)),
            pl.BlockSpec((R3, 128), lambda i: (i, 0)),
        ),
        scratch_shapes=(
            [pltpu.VMEM((R3 + 16, 32), bf16)] * 4 +
            [pltpu.VMEM((R3, 256), bf16),
             pltpu.VMEM((R3 + 16, 448), bf16),
             pltpu.VMEM((R3 + 8, 224), bf16)]
        ),
        compiler_params=pltpu.CompilerParams(
            dimension_semantics=("parallel",),
            vmem_limit_bytes=56 * 1024 * 1024),
    )(*xs, w1, b1, w2, b2t, wf, blt)
    return feat_k, logit_k


@jax.jit
def kernel(x, a1, b1t, a2, b2t, wlp, blt):
    n = x.shape[0]
    x2d = x.reshape(n, 28, 28).astype(jnp.bfloat16)
    xs = [x2d[:, m::4, :].reshape(n * 7, 28) for m in range(4)]

    a1blk = jnp.pad(a1[:, 2:30, :], ((0, 0), (0, 4), (0, 0)))
    w1 = jnp.zeros((8, 32, 4, 448), jnp.float32)
    for p, m in enumerate(_MORD):
        for i in range(5):
            s = _SRCS.index(((m + i - 2) % 4, (m + i - 2) // 4))
            w1 = w1.at[s, :, p, :].set(a1blk[i])
    w1 = w1.reshape(256, 4 * 448).astype(jnp.bfloat16)
    b1 = jnp.tile(b1t, (1, 4))

    t = a2[:, 32:256, :].reshape(5, 14, 16, 448)
    zb = jnp.zeros((14, 16, 448), jnp.float32)
    blocks = []
    for v in range(2):
        for e in (-1, 0, 1):
            ie, io = 2 * e + 2 - v, 2 * e + 3 - v
            even = t[ie] if 0 <= ie <= 4 else zb
            odd = t[io] if 0 <= io <= 4 else zb
            blocks.append(jnp.concatenate([even, odd], 1).reshape(448, 448))
    w2 = jnp.stack(blocks).astype(jnp.bfloat16)

    wf = wlp.astype(jnp.bfloat16)
    feat_k, logit_k = _forward(xs, w1, b1, w2, b2t, wf, blt)
    feat = feat_k.reshape(n, 7, 7, 32).transpose(0, 3, 1, 2).reshape(n, 1568)
    logits = logit_k[0::7, :10]
    return logits, feat


# speedup vs baseline: 10.7982x; 1.1643x over previous
import jax
import jax.numpy as jnp
from jax.experimental import pallas as pl
from jax.experimental.pallas import tpu as pltpu

_D = 8

_SRCS = [(2, -1), (3, -1), (0, 0), (1, 0), (2, 0), (3, 0), (0, 1), (1, 1)]
_MORD = [0, 2, 1, 3]


def _make_kernel(B):
    R3 = B * 7

    def body(xb_ref,
             w1_ref, b1_ref, w2_ref, b2t_ref, wf_ref, blt_ref,
             feat_ref, logit_ref, xq, fsp):
        f32 = jnp.float32
        bf16 = jnp.bfloat16
        h7 = jax.lax.broadcasted_iota(jnp.int32, (R3, 1), 0) % 7
        l448 = jax.lax.broadcasted_iota(jnp.int32, (1, 448), 1)

        def shifted(src, e):
            s = src[_D + e:_D + e + R3, :]
            if e < 0:
                s = jnp.where(h7 >= -e, s, jnp.bfloat16(0))
            elif e > 0:
                s = jnp.where(h7 <= 6 - e, s, jnp.bfloat16(0))
            return s

        acc1 = jnp.dot(xb_ref[...], w1_ref[...], preferred_element_type=f32)
        ybf = jnp.maximum(acc1 + b1_ref[...], 0.0).astype(bf16)

        hpe = jnp.maximum(ybf[:, 0:448], ybf[:, 896:1344])
        hpo = jnp.maximum(ybf[:, 448:896], ybf[:, 1344:1792])
        z16 = jnp.zeros((R3, 16), bf16)
        me = jnp.maximum(hpe, jnp.concatenate([hpe[:, 16:], z16], 1))
        mo = jnp.maximum(hpo, jnp.concatenate([hpo[:, 16:], z16], 1))
        mo_r = jnp.concatenate([z16, mo[:, :432]], 1)
        xq[0:_D, :] = jnp.zeros((_D, 448), bf16)
        xq[_D + R3:, :] = jnp.zeros((8, 448), bf16)
        xq[_D:_D + R3, :] = jnp.where(l448 % 32 < 16, me, mo_r)

        sl = [shifted(xq, e) for e in (-1, 0, 1)]
        y2 = []
        for v in range(2):
            acc = None
            for ei in range(3):
                p = jnp.dot(sl[ei], w2_ref[3 * v + ei],
                            preferred_element_type=f32)
                acc = p if acc is None else acc + p
            y2.append(jnp.maximum(acc + b2t_ref[...], 0.0))

        hq = jnp.maximum(y2[0], y2[1])
        fsp[R3:, :] = jnp.zeros_like(fsp[R3:, :])
        for q in range(7):
            part = jnp.maximum(hq[:, 64 * q:64 * q + 32],
                               hq[:, 64 * q + 32:64 * q + 64])
            feat_ref[:, 32 * q:32 * q + 32] = part
            fsp[0:R3, 32 * q:32 * q + 32] = part.astype(bf16)

        acc = None
        for h in range(7):
            p = jnp.dot(fsp[h:h + R3, :], wf_ref[h],
                        preferred_element_type=f32)
            acc = p if acc is None else acc + p
        logit_ref[...] = acc + blt_ref[...]

    return body, R3


def _forward(xb, w1, b1, w2, b2t, wf, blt):
    n = xb.shape[0] // 7
    B = 128 if n % 128 == 0 else (64 if n % 64 == 0 else n)
    body, R3 = _make_kernel(B)
    bf16 = jnp.bfloat16

    feat_k, logit_k = pl.pallas_call(
        body,
        out_shape=(jax.ShapeDtypeStruct((n * 7, 224), jnp.float32),
                   jax.ShapeDtypeStruct((n * 7, 128), jnp.float32)),
        grid=(n // B,),
        in_specs=[
            pl.BlockSpec((R3, 256), lambda i: (i, 0)),
            pl.BlockSpec((256, 1792), lambda i: (0, 0)),
            pl.BlockSpec((1, 1792), lambda i: (0, 0)),
            pl.BlockSpec((6, 448, 448), lambda i: (0, 0, 0)),
            pl.BlockSpec((1, 448), lambda i: (0, 0)),
            pl.BlockSpec((7, 224, 128), lambda i: (0, 0, 0)),
            pl.BlockSpec((1, 128), lambda i: (0, 0)),
        ],
        out_specs=(
            pl.BlockSpec((R3, 224), lambda i: (i, 0)),
            pl.BlockSpec((R3, 128), lambda i: (i, 0)),
        ),
        scratch_shapes=(
            [pltpu.VMEM((R3 + 16, 448), bf16),
             pltpu.VMEM((R3 + 8, 224), bf16)]
        ),
        compiler_params=pltpu.CompilerParams(
            dimension_semantics=("parallel",),
            vmem_limit_bytes=56 * 1024 * 1024),
    )(xb, w1, b1, w2, b2t, wf, blt)
    return feat_k, logit_k


@jax.jit
def kernel(x, a1, b1t, a2, b2t, wlp, blt):
    n = x.shape[0]
    x2d = x.reshape(n, 28, 28).astype(jnp.bfloat16)
    xs = [x2d[:, m::4, :] for m in range(4)]

    zrow = jnp.zeros((n, 1, 28), jnp.bfloat16)
    pieces = []
    for c, e in _SRCS:
        if e == -1:
            p = jnp.concatenate([zrow, xs[c][:, :6, :]], 1)
        elif e == 1:
            p = jnp.concatenate([xs[c][:, 1:, :], zrow], 1)
        else:
            p = xs[c]
        pieces.append(jnp.pad(p, ((0, 0), (0, 0), (0, 4))))
    xb = jnp.concatenate(pieces, 2).reshape(n * 7, 256)

    a1blk = jnp.pad(a1[:, 2:30, :], ((0, 0), (0, 4), (0, 0)))
    w1 = jnp.zeros((8, 32, 4, 448), jnp.float32)
    for p, m in enumerate(_MORD):
        for i in range(5):
            s = _SRCS.index(((m + i - 2) % 4, (m + i - 2) // 4))
            w1 = w1.at[s, :, p, :].set(a1blk[i])
    w1 = w1.reshape(256, 4 * 448).astype(jnp.bfloat16)
    b1 = jnp.tile(b1t, (1, 4))

    t = a2[:, 32:256, :].reshape(5, 14, 16, 448)
    zb = jnp.zeros((14, 16, 448), jnp.float32)
    blocks = []
    for v in range(2):
        for e in (-1, 0, 1):
            ie, io = 2 * e + 2 - v, 2 * e + 3 - v
            even = t[ie] if 0 <= ie <= 4 else zb
            odd = t[io] if 0 <= io <= 4 else zb
            blocks.append(jnp.concatenate([even, odd], 1).reshape(448, 448))
    w2 = jnp.stack(blocks).astype(jnp.bfloat16)

    wf = wlp.astype(jnp.bfloat16)
    feat_k, logit_k = _forward(xb, w1, b1, w2, b2t, wf, blt)
    feat = feat_k.reshape(n, 7, 7, 32).transpose(0, 3, 1, 2).reshape(n, 1568)
    logits = logit_k[0::7, :10]
    return logits, feat


# speedup vs baseline: 11.3908x; 1.0549x over previous
import jax
import jax.numpy as jnp
from jax.experimental import pallas as pl
from jax.experimental.pallas import tpu as pltpu

_D = 8

_SRCS = [(2, -1), (3, -1), (0, 0), (1, 0), (2, 0), (3, 0), (0, 1), (1, 1)]
_MORD = [0, 2, 1, 3]


def _make_kernel(B):
    R3 = B * 7

    def body(xb_ref,
             w1_ref, b1_ref, w2_ref, b2t_ref, wf_ref, blt_ref,
             feat_ref, logit_ref, xq, fsp):
        f32 = jnp.float32
        bf16 = jnp.bfloat16
        h7 = jax.lax.broadcasted_iota(jnp.int32, (R3, 1), 0) % 7
        l448 = jax.lax.broadcasted_iota(jnp.int32, (1, 448), 1)

        def shifted(src, e):
            s = src[_D + e:_D + e + R3, :]
            if e < 0:
                s = jnp.where(h7 >= -e, s, jnp.bfloat16(0))
            elif e > 0:
                s = jnp.where(h7 <= 6 - e, s, jnp.bfloat16(0))
            return s

        acc1 = jnp.dot(xb_ref[...], w1_ref[...], preferred_element_type=f32)
        ybf = jnp.maximum(acc1 + b1_ref[...], 0.0).astype(bf16)

        hpe = jnp.maximum(ybf[:, 0:448], ybf[:, 896:1344])
        hpo = jnp.maximum(ybf[:, 448:896], ybf[:, 1344:1792])
        z16 = jnp.zeros((R3, 16), bf16)
        mask16 = l448 % 32 < 16
        pe = jnp.concatenate([hpe[:, 16:], z16], 1)
        qo = jnp.concatenate([z16, hpo[:, :432]], 1)
        xq[0:_D, :] = jnp.zeros((_D, 448), bf16)
        xq[_D + R3:, :] = jnp.zeros((8, 448), bf16)
        xq[_D:_D + R3, :] = jnp.maximum(jnp.where(mask16, hpe, qo),
                                        jnp.where(mask16, pe, hpo))

        sl = [shifted(xq, e) for e in (-1, 0, 1)]
        y2 = []
        for v in range(2):
            acc = None
            for ei in range(3):
                p = jnp.dot(sl[ei], w2_ref[3 * v + ei],
                            preferred_element_type=f32)
                acc = p if acc is None else acc + p
            y2.append(jnp.maximum(acc + b2t_ref[...], 0.0))

        hq = jnp.maximum(y2[0], y2[1])
        mq = jnp.maximum(hq, jnp.concatenate(
            [hq[:, 32:], jnp.zeros((R3, 32), f32)], 1))
        fv = jnp.concatenate([mq[:, 64 * q:64 * q + 32] for q in range(7)], 1)
        feat_ref[...] = fv
        fsp[R3:, :] = jnp.zeros_like(fsp[R3:, :])
        fsp[0:R3, :] = fv.astype(bf16)

        acc = None
        for h in range(7):
            p = jnp.dot(fsp[h:h + R3, :], wf_ref[h],
                        preferred_element_type=f32)
            acc = p if acc is None else acc + p
        logit_ref[...] = acc + blt_ref[...]

    return body, R3


def _forward(xb, w1, b1, w2, b2t, wf, blt):
    n = xb.shape[0] // 7
    B = 128 if n % 128 == 0 else (64 if n % 64 == 0 else n)
    body, R3 = _make_kernel(B)
    bf16 = jnp.bfloat16

    feat_k, logit_k = pl.pallas_call(
        body,
        out_shape=(jax.ShapeDtypeStruct((n * 7, 224), jnp.float32),
                   jax.ShapeDtypeStruct((n * 7, 128), jnp.float32)),
        grid=(n // B,),
        in_specs=[
            pl.BlockSpec((R3, 256), lambda i: (i, 0)),
            pl.BlockSpec((256, 1792), lambda i: (0, 0)),
            pl.BlockSpec((1, 1792), lambda i: (0, 0)),
            pl.BlockSpec((6, 448, 448), lambda i: (0, 0, 0)),
            pl.BlockSpec((1, 448), lambda i: (0, 0)),
            pl.BlockSpec((7, 224, 128), lambda i: (0, 0, 0)),
            pl.BlockSpec((1, 128), lambda i: (0, 0)),
        ],
        out_specs=(
            pl.BlockSpec((R3, 224), lambda i: (i, 0)),
            pl.BlockSpec((R3, 128), lambda i: (i, 0)),
        ),
        scratch_shapes=(
            [pltpu.VMEM((R3 + 16, 448), bf16),
             pltpu.VMEM((R3 + 8, 224), bf16)]
        ),
        compiler_params=pltpu.CompilerParams(
            dimension_semantics=("parallel",),
            vmem_limit_bytes=56 * 1024 * 1024),
    )(xb, w1, b1, w2, b2t, wf, blt)
    return feat_k, logit_k


@jax.jit
def kernel(x, a1, b1t, a2, b2t, wlp, blt):
    n = x.shape[0]
    x2d = x.reshape(n, 28, 28).astype(jnp.bfloat16)
    xs = [x2d[:, m::4, :] for m in range(4)]

    zrow = jnp.zeros((n, 1, 28), jnp.bfloat16)
    pieces = []
    for c, e in _SRCS:
        if e == -1:
            p = jnp.concatenate([zrow, xs[c][:, :6, :]], 1)
        elif e == 1:
            p = jnp.concatenate([xs[c][:, 1:, :], zrow], 1)
        else:
            p = xs[c]
        pieces.append(jnp.pad(p, ((0, 0), (0, 0), (0, 4))))
    xb = jnp.concatenate(pieces, 2).reshape(n * 7, 256)

    a1blk = jnp.pad(a1[:, 2:30, :], ((0, 0), (0, 4), (0, 0)))
    w1 = jnp.zeros((8, 32, 4, 448), jnp.float32)
    for p, m in enumerate(_MORD):
        for i in range(5):
            s = _SRCS.index(((m + i - 2) % 4, (m + i - 2) // 4))
            w1 = w1.at[s, :, p, :].set(a1blk[i])
    w1 = w1.reshape(256, 4 * 448).astype(jnp.bfloat16)
    b1 = jnp.tile(b1t, (1, 4))

    t = a2[:, 32:256, :].reshape(5, 14, 16, 448)
    zb = jnp.zeros((14, 16, 448), jnp.float32)
    blocks = []
    for v in range(2):
        for e in (-1, 0, 1):
            ie, io = 2 * e + 2 - v, 2 * e + 3 - v
            even = t[ie] if 0 <= ie <= 4 else zb
            odd = t[io] if 0 <= io <= 4 else zb
            blocks.append(jnp.concatenate([even, odd], 1).reshape(448, 448))
    w2 = jnp.stack(blocks).astype(jnp.bfloat16)

    wf = wlp.astype(jnp.bfloat16)
    feat_k, logit_k = _forward(xb, w1, b1, w2, b2t, wf, blt)
    feat = feat_k.reshape(n, 7, 7, 32).transpose(0, 3, 1, 2).reshape(n, 1568)
    logits = logit_k[0::7, :10]
    return logits, feat
